# baseline (speedup 1.0000x reference)
import functools

import jax
import jax.numpy as jnp
from jax import lax
from jax.experimental import pallas as pl
from jax.experimental.pallas import tpu as pltpu
from jax.experimental.pallas import tpu_sc as plsc

N_NODES = 10000
N_EDGES = 320000
HD = 128
NL = 4
NA = 100

NP = 10240
EP = 327680
NC = 2
NS = 16
NW = NC * NS
EDGES_W = EP // NW
ECHUNK = 128
ESTEPS = EDGES_W // ECHUNK
ROWS_T = NP // NS
ZCH = ECHUNK
NBUF = 2
GSTEPS = ESTEPS // NBUF
EROWS = EP // ECHUNK


def _zero_vmem_rows(ref, nrows):
    zv = jnp.zeros((16,), jnp.float32)

    def body(i, _):
        r = i // (HD // 16)
        c = (i % (HD // 16)) * 16
        ref[r, pl.ds(c, 16)] = zv
        return 0

    lax.fori_loop(0, nrows * (HD // 16), body, 0)


def _sc_agg_impl(with_deg, x_hbm, src_hbm, dst_hbm, m_hbm, *rest):
    if with_deg:
        degp_hbm = rest[0]
        rest = rest[1:]
    sbuf = rest[0:NBUF]
    dbuf = rest[NBUF:2 * NBUF]
    rows = rest[2 * NBUF:3 * NBUF]
    acc = rest[3 * NBUF]
    isem = rest[3 * NBUF + 1:4 * NBUF + 1]
    gsem = rest[4 * NBUF + 1:5 * NBUF + 1]
    ssem = rest[5 * NBUF + 1:6 * NBUF + 1]
    if with_deg:
        ones_v = rest[6 * NBUF + 1]
        degstage = rest[6 * NBUF + 2]
        deg_acc = rest[6 * NBUF + 3]
        dsem = rest[6 * NBUF + 4:7 * NBUF + 4]
    r0 = rows[0]
    c = lax.axis_index("c")
    s = lax.axis_index("s")
    wid = c * NS + s
    cbase = wid * ESTEPS

    def fire_idx(b, t):
        pltpu.async_copy(src_hbm.at[cbase + t], sbuf[b], isem[b])
        pltpu.async_copy(dst_hbm.at[cbase + t], dbuf[b], isem[b])

    def wait_idx(b, t):
        pltpu.make_async_copy(src_hbm.at[cbase + t], sbuf[b], isem[b]).wait()
        pltpu.make_async_copy(dst_hbm.at[cbase + t], dbuf[b], isem[b]).wait()

    def fire_gather(b):
        pltpu.async_copy(x_hbm.at[sbuf[b]], rows[b], gsem[b])

    def wait_gather(b):
        pltpu.make_async_copy(x_hbm.at[sbuf[b]], rows[b], gsem[b]).wait()

    def fire_scatter(b):
        pltpu.async_copy(rows[b], acc.at[dbuf[b]], ssem[b], add=True)
        if with_deg:
            pltpu.async_copy(ones_v, deg_acc.at[dbuf[b]], dsem[b], add=True)

    def wait_scatter(b):
        pltpu.make_async_copy(rows[b], acc.at[dbuf[b]], ssem[b]).wait()
        if with_deg:
            pltpu.make_async_copy(
                ones_v, deg_acc.at[dbuf[b]], dsem[b]).wait()

    for b in range(NBUF):
        fire_idx(b, b)

    _zero_vmem_rows(r0, ZCH)
    for k in range(ROWS_T // ZCH):
        pltpu.sync_copy(r0, acc.at[pl.ds(s * ROWS_T + k * ZCH, ZCH)])
    if with_deg:
        ones16 = jnp.ones((16,), jnp.float32)
        zero16 = jnp.zeros((16,), jnp.float32)

        def fill(i, _):
            ones_v[pl.ds(i * 16, 16)] = ones16
            return 0
        lax.fori_loop(0, ECHUNK // 16, fill, 0)

        def zfill(i, _):
            degstage[pl.ds(i * 16, 16)] = zero16
            return 0
        lax.fori_loop(0, ROWS_T // 16, zfill, 0)
        pltpu.sync_copy(degstage, deg_acc.at[pl.ds(s * ROWS_T, ROWS_T)])
    plsc.subcore_barrier()

    for b in range(NBUF):
        wait_idx(b, b)
        fire_gather(b)

    def gbody(g, _):
        for b in range(NBUF):
            wait_gather(b)
            fire_scatter(b)

        @pl.when(g < GSTEPS - 1)
        def _():
            t2 = (g + 1) * NBUF
            for b in range(NBUF):
                wait_scatter(b)
                fire_idx(b, t2 + b)
            for b in range(NBUF):
                wait_idx(b, t2 + b)
                fire_gather(b)
        return 0
    lax.fori_loop(0, GSTEPS, gbody, 0)

    for b in range(NBUF):
        wait_scatter(b)
    plsc.subcore_barrier()

    for k in range(ROWS_T // ZCH):
        r = s * ROWS_T + k * ZCH
        pltpu.sync_copy(acc.at[pl.ds(r, ZCH)], r0)
        pltpu.sync_copy(r0, m_hbm.at[c, pl.ds(r, ZCH)])
    if with_deg:
        pltpu.sync_copy(deg_acc.at[pl.ds(s * ROWS_T, ROWS_T)], degstage)
        pltpu.sync_copy(degstage, degp_hbm.at[c, pl.ds(s * ROWS_T, ROWS_T)])


def _make_sc_agg(with_deg):
    outs = [jax.ShapeDtypeStruct((NC, NP, HD), jnp.float32)]
    scratch = (
        [pltpu.VMEM((ECHUNK,), jnp.int32)] * (2 * NBUF)
        + [pltpu.VMEM((ECHUNK, HD), jnp.float32)] * NBUF
        + [pltpu.VMEM_SHARED((NP, HD), jnp.float32)]
        + [pltpu.SemaphoreType.DMA] * (3 * NBUF)
    )
    if with_deg:
        outs.append(jax.ShapeDtypeStruct((NC, NP), jnp.float32))
        scratch = scratch + (
            [pltpu.VMEM((ECHUNK,), jnp.float32),
             pltpu.VMEM((ROWS_T,), jnp.float32),
             pltpu.VMEM_SHARED((NP,), jnp.float32)]
            + [pltpu.SemaphoreType.DMA] * NBUF
        )

    @functools.partial(
        pl.kernel,
        out_type=tuple(outs) if with_deg else outs[0],
        mesh=plsc.VectorSubcoreMesh(core_axis_name="c", subcore_axis_name="s"),
        scratch_types=scratch,
    )
    def agg(*refs):
        _sc_agg_impl(with_deg, *refs)

    return agg


_sc_agg = _make_sc_agg(False)
_sc_agg_deg = _make_sc_agg(True)


ROWS_B = 2048
GRID = NP // ROWS_B
_BN_SCALE = 1.0 / (1.0 + 1e-5) ** 0.5


def _layer_math(x, m0, m1, d0, d1, w, b, gamma, beta):
    inv = 1.0 / jnp.maximum(d0 + d1, 1.0)
    cagg = (m0 + m1) * inv
    t = (jnp.dot(x, w[:HD, :], preferred_element_type=jnp.float32)
         + jnp.dot(cagg, w[HD:, :], preferred_element_type=jnp.float32) + b)
    nrm = jnp.sqrt(jnp.sum(t * t, axis=1, keepdims=True))
    t = t / jnp.maximum(nrm, 1e-12)
    t = jnp.maximum(t, 0.0)
    t = t * (gamma * _BN_SCALE) + beta
    return x + t


def _tc_embed_kernel(h_ref, emb_ref, o_ref):
    oh = (h_ref[...] == lax.broadcasted_iota(jnp.int32, (1, NA), 1))
    o_ref[...] = jnp.dot(oh.astype(jnp.float32), emb_ref[...],
                         preferred_element_type=jnp.float32)


def _tc_layer_kernel(x_ref, m0_ref, m1_ref, d0_ref, d1_ref,
                     w_ref, b_ref, g_ref, bt_ref, o_ref):
    o_ref[...] = _layer_math(x_ref[...], m0_ref[...], m1_ref[...],
                             d0_ref[...], d1_ref[...], w_ref[...],
                             b_ref[...], g_ref[...], bt_ref[...])


def _tc_final_kernel(x_ref, m0_ref, m1_ref, d0_ref, d1_ref,
                     w_ref, b_ref, g_ref, bt_ref,
                     mw1_ref, mb1_ref, mw2_ref, mb2_ref, mw3_ref, mb3_ref,
                     y_ref, acc_ref):
    i = pl.program_id(0)
    xn = _layer_math(x_ref[...], m0_ref[...], m1_ref[...],
                     d0_ref[...], d1_ref[...], w_ref[...],
                     b_ref[...], g_ref[...], bt_ref[...])
    rows = i * ROWS_B + lax.broadcasted_iota(jnp.int32, (ROWS_B, 1), 0)
    xn = jnp.where(rows < N_NODES, xn, 0.0)
    part = jnp.sum(xn, axis=0, keepdims=True)

    @pl.when(i == 0)
    def _():
        acc_ref[...] = jnp.zeros_like(acc_ref)

    acc_ref[...] += part

    @pl.when(i == GRID - 1)
    def _():
        hg = acc_ref[...] * (1.0 / N_NODES)
        y = jnp.maximum(jnp.dot(hg, mw1_ref[...],
                                preferred_element_type=jnp.float32)
                        + mb1_ref[...], 0.0)
        y = jnp.maximum(jnp.dot(y, mw2_ref[...],
                                preferred_element_type=jnp.float32)
                        + mb2_ref[...], 0.0)
        y_ref[...] = (jnp.dot(y, mw3_ref[...],
                              preferred_element_type=jnp.float32)
                      + mb3_ref[...])


_row_spec = pl.BlockSpec((ROWS_B, HD), lambda i: (i, 0))
_deg_spec = pl.BlockSpec((ROWS_B, 1), lambda i: (i, 0))


def _whole(shape):
    return pl.BlockSpec(shape, lambda i, _s=shape: tuple(0 for _ in _s))


def _tc_embed(h2, emb):
    return pl.pallas_call(
        _tc_embed_kernel,
        grid=(GRID,),
        in_specs=[_deg_spec, _whole((NA, HD))],
        out_specs=_row_spec,
        out_shape=jax.ShapeDtypeStruct((NP, HD), jnp.float32),
    )(h2, emb)


def _tc_layer(x, m0, m1, d0, d1, w, b, g, bt):
    return pl.pallas_call(
        _tc_layer_kernel,
        grid=(GRID,),
        in_specs=[_row_spec, _row_spec, _row_spec, _deg_spec, _deg_spec,
                  _whole((2 * HD, HD)), _whole((1, HD)), _whole((1, HD)),
                  _whole((1, HD))],
        out_specs=_row_spec,
        out_shape=jax.ShapeDtypeStruct((NP, HD), jnp.float32),
    )(x, m0, m1, d0, d1, w, b, g, bt)


def _tc_final(x, m0, m1, d0, d1, w, b, g, bt, mw1, mb1, mw2, mb2, mw3, mb3):
    return pl.pallas_call(
        _tc_final_kernel,
        grid=(GRID,),
        in_specs=[_row_spec, _row_spec, _row_spec, _deg_spec, _deg_spec,
                  _whole((2 * HD, HD)), _whole((1, HD)), _whole((1, HD)),
                  _whole((1, HD)),
                  _whole((HD, HD // 2)), _whole((1, HD // 2)),
                  _whole((HD // 2, HD // 4)), _whole((1, HD // 4)),
                  _whole((HD // 4, 1)), _whole((1, 1))],
        out_specs=_whole((1, 1)),
        out_shape=jax.ShapeDtypeStruct((1, 1), jnp.float32),
        scratch_shapes=[pltpu.VMEM((1, HD), jnp.float32)],
    )(x, m0, m1, d0, d1, w, b, g, bt, mw1, mb1, mw2, mb2, mw3, mb3)


def kernel(edge_index, h, e, emb, W, b, gamma, beta,
           mW1, mb1, mW2, mb2, mW3, mb3):
    del e
    src = edge_index[0].astype(jnp.int32)
    dst = edge_index[1].astype(jnp.int32)
    pad = N_NODES + (jnp.arange(EP - N_EDGES, dtype=jnp.int32)
                     % (NP - N_NODES))
    src_pad = jnp.concatenate([src, pad]).reshape(EROWS, ECHUNK)
    dst_pad = jnp.concatenate([dst, pad]).reshape(EROWS, ECHUNK)
    h2 = jnp.concatenate(
        [h.astype(jnp.int32),
         jnp.zeros((NP - N_NODES,), jnp.int32)]).reshape(NP, 1)

    x = _tc_embed(h2, emb)

    b2 = b.reshape(NL, 1, HD)
    g2 = gamma.reshape(NL, 1, HD)
    bt2 = beta.reshape(NL, 1, HD)

    m, degp = _sc_agg_deg(x, src_pad, dst_pad)
    d0 = degp[0].reshape(NP, 1)
    d1 = degp[1].reshape(NP, 1)
    x = _tc_layer(x, m[0], m[1], d0, d1, W[0], b2[0], g2[0], bt2[0])
    for l in range(1, NL - 1):
        m = _sc_agg(x, src_pad, dst_pad)
        x = _tc_layer(x, m[0], m[1], d0, d1, W[l], b2[l], g2[l], bt2[l])
    m = _sc_agg(x, src_pad, dst_pad)
    y = _tc_final(x, m[0], m[1], d0, d1, W[NL - 1], b2[NL - 1],
                  g2[NL - 1], bt2[NL - 1],
                  mW1, mb1.reshape(1, HD // 2), mW2, mb2.reshape(1, HD // 4),
                  mW3, mb3.reshape(1, 1))
    return y


# speedup vs baseline: 1.1247x; 1.1247x over previous
import functools

import jax
import jax.numpy as jnp
from jax import lax
from jax.experimental import pallas as pl
from jax.experimental.pallas import tpu as pltpu
from jax.experimental.pallas import tpu_sc as plsc

N_NODES = 10000
N_EDGES = 320000
HD = 128
NL = 4
NA = 100

NP = 10240
EP = 327680
NC = 2
NS = 16
NW = NC * NS
EDGES_W = EP // NW
ECHUNK = 80
ESTEPS = EDGES_W // ECHUNK
ROWS_T = NP // NS
ZCH = ECHUNK
NBUF = 4
GSTEPS = ESTEPS // NBUF
EROWS = EP // ECHUNK


def _zero_vmem_rows(ref, nrows):
    zv = jnp.zeros((16,), jnp.float32)

    def body(i, _):
        r = i // (HD // 16)
        c = (i % (HD // 16)) * 16
        ref[r, pl.ds(c, 16)] = zv
        return 0

    lax.fori_loop(0, nrows * (HD // 16), body, 0)


def _sc_agg_impl(with_deg, x_hbm, src_hbm, dst_hbm, m_hbm, *rest):
    if with_deg:
        degp_hbm = rest[0]
        rest = rest[1:]
    sbuf = rest[0:NBUF]
    dbuf = rest[NBUF:2 * NBUF]
    rows = rest[2 * NBUF:3 * NBUF]
    acc = rest[3 * NBUF]
    isem = rest[3 * NBUF + 1:4 * NBUF + 1]
    gsem = rest[4 * NBUF + 1:5 * NBUF + 1]
    ssem = rest[5 * NBUF + 1:6 * NBUF + 1]
    if with_deg:
        ones_v = rest[6 * NBUF + 1]
        degstage = rest[6 * NBUF + 2]
        deg_acc = rest[6 * NBUF + 3]
        dsem = rest[6 * NBUF + 4:7 * NBUF + 4]
    r0 = rows[0]
    c = lax.axis_index("c")
    s = lax.axis_index("s")
    wid = c * NS + s
    cbase = wid * ESTEPS

    def fire_idx(b, t):
        pltpu.async_copy(src_hbm.at[cbase + t], sbuf[b], isem[b])
        pltpu.async_copy(dst_hbm.at[cbase + t], dbuf[b], isem[b])

    def wait_idx(b, t):
        pltpu.make_async_copy(src_hbm.at[cbase + t], sbuf[b], isem[b]).wait()
        pltpu.make_async_copy(dst_hbm.at[cbase + t], dbuf[b], isem[b]).wait()

    def fire_gather(b):
        pltpu.async_copy(x_hbm.at[sbuf[b]], rows[b], gsem[b])

    def wait_gather(b):
        pltpu.make_async_copy(x_hbm.at[sbuf[b]], rows[b], gsem[b]).wait()

    def fire_scatter(b):
        pltpu.async_copy(rows[b], acc.at[dbuf[b]], ssem[b], add=True)
        if with_deg:
            pltpu.async_copy(ones_v, deg_acc.at[dbuf[b]], dsem[b], add=True)

    def wait_scatter(b):
        pltpu.make_async_copy(rows[b], acc.at[dbuf[b]], ssem[b]).wait()
        if with_deg:
            pltpu.make_async_copy(
                ones_v, deg_acc.at[dbuf[b]], dsem[b]).wait()

    for b in range(NBUF):
        fire_idx(b, b)

    _zero_vmem_rows(r0, ZCH)
    for k in range(ROWS_T // ZCH):
        pltpu.sync_copy(r0, acc.at[pl.ds(s * ROWS_T + k * ZCH, ZCH)])
    if with_deg:
        ones16 = jnp.ones((16,), jnp.float32)
        zero16 = jnp.zeros((16,), jnp.float32)

        def fill(i, _):
            ones_v[pl.ds(i * 16, 16)] = ones16
            return 0
        lax.fori_loop(0, ECHUNK // 16, fill, 0)

        def zfill(i, _):
            degstage[pl.ds(i * 16, 16)] = zero16
            return 0
        lax.fori_loop(0, ROWS_T // 16, zfill, 0)
        pltpu.sync_copy(degstage, deg_acc.at[pl.ds(s * ROWS_T, ROWS_T)])
    plsc.subcore_barrier()

    for b in range(NBUF):
        wait_idx(b, b)
        fire_gather(b)

    def gbody(g, _):
        for b in range(NBUF):
            wait_gather(b)
            fire_scatter(b)

        @pl.when(g < GSTEPS - 1)
        def _():
            t2 = (g + 1) * NBUF
            for b in range(NBUF):
                wait_scatter(b)
                fire_idx(b, t2 + b)
            for b in range(NBUF):
                wait_idx(b, t2 + b)
                fire_gather(b)
        return 0
    lax.fori_loop(0, GSTEPS, gbody, 0)

    for b in range(NBUF):
        wait_scatter(b)
    plsc.subcore_barrier()

    for k in range(ROWS_T // ZCH):
        r = s * ROWS_T + k * ZCH
        pltpu.sync_copy(acc.at[pl.ds(r, ZCH)], r0)
        pltpu.sync_copy(r0, m_hbm.at[c, pl.ds(r, ZCH)])
    if with_deg:
        pltpu.sync_copy(deg_acc.at[pl.ds(s * ROWS_T, ROWS_T)], degstage)
        pltpu.sync_copy(degstage, degp_hbm.at[c, pl.ds(s * ROWS_T, ROWS_T)])


def _make_sc_agg(with_deg):
    outs = [jax.ShapeDtypeStruct((NC, NP, HD), jnp.float32)]
    scratch = (
        [pltpu.VMEM((ECHUNK,), jnp.int32)] * (2 * NBUF)
        + [pltpu.VMEM((ECHUNK, HD), jnp.float32)] * NBUF
        + [pltpu.VMEM_SHARED((NP, HD), jnp.float32)]
        + [pltpu.SemaphoreType.DMA] * (3 * NBUF)
    )
    if with_deg:
        outs.append(jax.ShapeDtypeStruct((NC, NP), jnp.float32))
        scratch = scratch + (
            [pltpu.VMEM((ECHUNK,), jnp.float32),
             pltpu.VMEM((ROWS_T,), jnp.float32),
             pltpu.VMEM_SHARED((NP,), jnp.float32)]
            + [pltpu.SemaphoreType.DMA] * NBUF
        )

    @functools.partial(
        pl.kernel,
        out_type=tuple(outs) if with_deg else outs[0],
        mesh=plsc.VectorSubcoreMesh(core_axis_name="c", subcore_axis_name="s"),
        scratch_types=scratch,
    )
    def agg(*refs):
        _sc_agg_impl(with_deg, *refs)

    return agg


_sc_agg = _make_sc_agg(False)
_sc_agg_deg = _make_sc_agg(True)


ROWS_B = 2048
GRID = NP // ROWS_B
_BN_SCALE = 1.0 / (1.0 + 1e-5) ** 0.5


def _layer_math(x, m0, m1, d0, d1, w, b, gamma, beta):
    inv = 1.0 / jnp.maximum(d0 + d1, 1.0)
    cagg = (m0 + m1) * inv
    t = (jnp.dot(x, w[:HD, :], preferred_element_type=jnp.float32)
         + jnp.dot(cagg, w[HD:, :], preferred_element_type=jnp.float32) + b)
    nrm = jnp.sqrt(jnp.sum(t * t, axis=1, keepdims=True))
    t = t / jnp.maximum(nrm, 1e-12)
    t = jnp.maximum(t, 0.0)
    t = t * (gamma * _BN_SCALE) + beta
    return x + t


def _tc_embed_kernel(h_ref, emb_ref, o_ref):
    oh = (h_ref[...] == lax.broadcasted_iota(jnp.int32, (1, NA), 1))
    o_ref[...] = jnp.dot(oh.astype(jnp.float32), emb_ref[...],
                         preferred_element_type=jnp.float32)


def _tc_layer_kernel(x_ref, m0_ref, m1_ref, d0_ref, d1_ref,
                     w_ref, b_ref, g_ref, bt_ref, o_ref):
    o_ref[...] = _layer_math(x_ref[...], m0_ref[...], m1_ref[...],
                             d0_ref[...], d1_ref[...], w_ref[...],
                             b_ref[...], g_ref[...], bt_ref[...])


def _tc_final_kernel(x_ref, m0_ref, m1_ref, d0_ref, d1_ref,
                     w_ref, b_ref, g_ref, bt_ref,
                     mw1_ref, mb1_ref, mw2_ref, mb2_ref, mw3_ref, mb3_ref,
                     y_ref, acc_ref):
    i = pl.program_id(0)
    xn = _layer_math(x_ref[...], m0_ref[...], m1_ref[...],
                     d0_ref[...], d1_ref[...], w_ref[...],
                     b_ref[...], g_ref[...], bt_ref[...])
    rows = i * ROWS_B + lax.broadcasted_iota(jnp.int32, (ROWS_B, 1), 0)
    xn = jnp.where(rows < N_NODES, xn, 0.0)
    part = jnp.sum(xn, axis=0, keepdims=True)

    @pl.when(i == 0)
    def _():
        acc_ref[...] = jnp.zeros_like(acc_ref)

    acc_ref[...] += part

    @pl.when(i == GRID - 1)
    def _():
        hg = acc_ref[...] * (1.0 / N_NODES)
        y = jnp.maximum(jnp.dot(hg, mw1_ref[...],
                                preferred_element_type=jnp.float32)
                        + mb1_ref[...], 0.0)
        y = jnp.maximum(jnp.dot(y, mw2_ref[...],
                                preferred_element_type=jnp.float32)
                        + mb2_ref[...], 0.0)
        y_ref[...] = (jnp.dot(y, mw3_ref[...],
                              preferred_element_type=jnp.float32)
                      + mb3_ref[...])


_row_spec = pl.BlockSpec((ROWS_B, HD), lambda i: (i, 0))
_deg_spec = pl.BlockSpec((ROWS_B, 1), lambda i: (i, 0))


def _whole(shape):
    return pl.BlockSpec(shape, lambda i, _s=shape: tuple(0 for _ in _s))


def _tc_embed(h2, emb):
    return pl.pallas_call(
        _tc_embed_kernel,
        grid=(GRID,),
        in_specs=[_deg_spec, _whole((NA, HD))],
        out_specs=_row_spec,
        out_shape=jax.ShapeDtypeStruct((NP, HD), jnp.float32),
    )(h2, emb)


def _tc_layer(x, m0, m1, d0, d1, w, b, g, bt):
    return pl.pallas_call(
        _tc_layer_kernel,
        grid=(GRID,),
        in_specs=[_row_spec, _row_spec, _row_spec, _deg_spec, _deg_spec,
                  _whole((2 * HD, HD)), _whole((1, HD)), _whole((1, HD)),
                  _whole((1, HD))],
        out_specs=_row_spec,
        out_shape=jax.ShapeDtypeStruct((NP, HD), jnp.float32),
    )(x, m0, m1, d0, d1, w, b, g, bt)


def _tc_final(x, m0, m1, d0, d1, w, b, g, bt, mw1, mb1, mw2, mb2, mw3, mb3):
    return pl.pallas_call(
        _tc_final_kernel,
        grid=(GRID,),
        in_specs=[_row_spec, _row_spec, _row_spec, _deg_spec, _deg_spec,
                  _whole((2 * HD, HD)), _whole((1, HD)), _whole((1, HD)),
                  _whole((1, HD)),
                  _whole((HD, HD // 2)), _whole((1, HD // 2)),
                  _whole((HD // 2, HD // 4)), _whole((1, HD // 4)),
                  _whole((HD // 4, 1)), _whole((1, 1))],
        out_specs=_whole((1, 1)),
        out_shape=jax.ShapeDtypeStruct((1, 1), jnp.float32),
        scratch_shapes=[pltpu.VMEM((1, HD), jnp.float32)],
    )(x, m0, m1, d0, d1, w, b, g, bt, mw1, mb1, mw2, mb2, mw3, mb3)


def kernel(edge_index, h, e, emb, W, b, gamma, beta,
           mW1, mb1, mW2, mb2, mW3, mb3):
    del e
    src = edge_index[0].astype(jnp.int32)
    dst = edge_index[1].astype(jnp.int32)
    pad = N_NODES + (jnp.arange(EP - N_EDGES, dtype=jnp.int32)
                     % (NP - N_NODES))
    src_pad = jnp.concatenate([src, pad]).reshape(EROWS, ECHUNK)
    dst_pad = jnp.concatenate([dst, pad]).reshape(EROWS, ECHUNK)
    h2 = jnp.concatenate(
        [h.astype(jnp.int32),
         jnp.zeros((NP - N_NODES,), jnp.int32)]).reshape(NP, 1)

    x = _tc_embed(h2, emb)

    b2 = b.reshape(NL, 1, HD)
    g2 = gamma.reshape(NL, 1, HD)
    bt2 = beta.reshape(NL, 1, HD)

    m, degp = _sc_agg_deg(x, src_pad, dst_pad)
    d0 = degp[0].reshape(NP, 1)
    d1 = degp[1].reshape(NP, 1)
    x = _tc_layer(x, m[0], m[1], d0, d1, W[0], b2[0], g2[0], bt2[0])
    for l in range(1, NL - 1):
        m = _sc_agg(x, src_pad, dst_pad)
        x = _tc_layer(x, m[0], m[1], d0, d1, W[l], b2[l], g2[l], bt2[l])
    m = _sc_agg(x, src_pad, dst_pad)
    y = _tc_final(x, m[0], m[1], d0, d1, W[NL - 1], b2[NL - 1],
                  g2[NL - 1], bt2[NL - 1],
                  mW1, mb1.reshape(1, HD // 2), mW2, mb2.reshape(1, HD // 4),
                  mW3, mb3.reshape(1, 1))
    return y


# speedup vs baseline: 1.1380x; 1.0119x over previous
import functools

import jax
import jax.numpy as jnp
from jax import lax
from jax.experimental import pallas as pl
from jax.experimental.pallas import tpu as pltpu
from jax.experimental.pallas import tpu_sc as plsc

N_NODES = 10000
N_EDGES = 320000
HD = 128
NL = 4
NA = 100

NP = 10240
EP = 327680
NC = 2
NS = 16
NW = NC * NS
EDGES_W = EP // NW
ECHUNK = 64
ESTEPS = EDGES_W // ECHUNK
ROWS_T = NP // NS
ZCH = ECHUNK
NBUF = 5
GSTEPS = ESTEPS // NBUF
EROWS = EP // ECHUNK


def _zero_vmem_rows(ref, nrows):
    zv = jnp.zeros((16,), jnp.float32)

    def body(i, _):
        r = i // (HD // 16)
        c = (i % (HD // 16)) * 16
        ref[r, pl.ds(c, 16)] = zv
        return 0

    lax.fori_loop(0, nrows * (HD // 16), body, 0)


def _sc_agg_impl(with_deg, x_hbm, src_hbm, dst_hbm, m_hbm, *rest):
    if with_deg:
        degp_hbm = rest[0]
        rest = rest[1:]
    sbuf = rest[0:NBUF]
    dbuf = rest[NBUF:2 * NBUF]
    rows = rest[2 * NBUF:3 * NBUF]
    acc = rest[3 * NBUF]
    isem = rest[3 * NBUF + 1:4 * NBUF + 1]
    gsem = rest[4 * NBUF + 1:5 * NBUF + 1]
    ssem = rest[5 * NBUF + 1:6 * NBUF + 1]
    if with_deg:
        ones_v = rest[6 * NBUF + 1]
        degstage = rest[6 * NBUF + 2]
        deg_acc = rest[6 * NBUF + 3]
        dsem = rest[6 * NBUF + 4:7 * NBUF + 4]
    r0 = rows[0]
    c = lax.axis_index("c")
    s = lax.axis_index("s")
    wid = c * NS + s
    cbase = wid * ESTEPS

    def fire_idx(b, t):
        pltpu.async_copy(src_hbm.at[cbase + t], sbuf[b], isem[b])
        pltpu.async_copy(dst_hbm.at[cbase + t], dbuf[b], isem[b])

    def wait_idx(b, t):
        pltpu.make_async_copy(src_hbm.at[cbase + t], sbuf[b], isem[b]).wait()
        pltpu.make_async_copy(dst_hbm.at[cbase + t], dbuf[b], isem[b]).wait()

    def fire_gather(b):
        pltpu.async_copy(x_hbm.at[sbuf[b]], rows[b], gsem[b])

    def wait_gather(b):
        pltpu.make_async_copy(x_hbm.at[sbuf[b]], rows[b], gsem[b]).wait()

    def fire_scatter(b):
        pltpu.async_copy(rows[b], acc.at[dbuf[b]], ssem[b], add=True)
        if with_deg:
            pltpu.async_copy(ones_v, deg_acc.at[dbuf[b]], dsem[b], add=True)

    def wait_scatter(b):
        pltpu.make_async_copy(rows[b], acc.at[dbuf[b]], ssem[b]).wait()
        if with_deg:
            pltpu.make_async_copy(
                ones_v, deg_acc.at[dbuf[b]], dsem[b]).wait()

    for b in range(NBUF):
        fire_idx(b, b)

    _zero_vmem_rows(r0, ZCH)
    nz = ROWS_T // ZCH
    for k in range(nz):
        pltpu.async_copy(r0, acc.at[pl.ds(s * ROWS_T + k * ZCH, ZCH)],
                         ssem[k % NBUF])
    if with_deg:
        ones16 = jnp.ones((16,), jnp.float32)
        zero16 = jnp.zeros((16,), jnp.float32)

        def fill(i, _):
            ones_v[pl.ds(i * 16, 16)] = ones16
            return 0
        lax.fori_loop(0, ECHUNK // 16, fill, 0)

        def zfill(i, _):
            degstage[pl.ds(i * 16, 16)] = zero16
            return 0
        lax.fori_loop(0, ROWS_T // 16, zfill, 0)
        pltpu.sync_copy(degstage, deg_acc.at[pl.ds(s * ROWS_T, ROWS_T)])
    for k in range(nz):
        pltpu.make_async_copy(
            r0, acc.at[pl.ds(s * ROWS_T + k * ZCH, ZCH)],
            ssem[k % NBUF]).wait()
    for b in range(NBUF):
        wait_idx(b, b)
        fire_gather(b)
    plsc.subcore_barrier()

    def gbody(g, _):
        for b in range(NBUF):
            wait_gather(b)
            fire_scatter(b)

        @pl.when(g < GSTEPS - 1)
        def _():
            t2 = (g + 1) * NBUF
            for b in range(NBUF):
                wait_scatter(b)
                fire_idx(b, t2 + b)
            for b in range(NBUF):
                wait_idx(b, t2 + b)
                fire_gather(b)
        return 0
    lax.fori_loop(0, GSTEPS, gbody, 0)

    for b in range(NBUF):
        wait_scatter(b)
    plsc.subcore_barrier()

    for k in range(ROWS_T // ZCH):
        r = s * ROWS_T + k * ZCH
        pltpu.sync_copy(acc.at[pl.ds(r, ZCH)], r0)
        pltpu.sync_copy(r0, m_hbm.at[c, pl.ds(r, ZCH)])
    if with_deg:
        pltpu.sync_copy(deg_acc.at[pl.ds(s * ROWS_T, ROWS_T)], degstage)
        pltpu.sync_copy(degstage, degp_hbm.at[c, pl.ds(s * ROWS_T, ROWS_T)])


def _make_sc_agg(with_deg):
    outs = [jax.ShapeDtypeStruct((NC, NP, HD), jnp.float32)]
    scratch = (
        [pltpu.VMEM((ECHUNK,), jnp.int32)] * (2 * NBUF)
        + [pltpu.VMEM((ECHUNK, HD), jnp.float32)] * NBUF
        + [pltpu.VMEM_SHARED((NP, HD), jnp.float32)]
        + [pltpu.SemaphoreType.DMA] * (3 * NBUF)
    )
    if with_deg:
        outs.append(jax.ShapeDtypeStruct((NC, NP), jnp.float32))
        scratch = scratch + (
            [pltpu.VMEM((ECHUNK,), jnp.float32),
             pltpu.VMEM((ROWS_T,), jnp.float32),
             pltpu.VMEM_SHARED((NP,), jnp.float32)]
            + [pltpu.SemaphoreType.DMA] * NBUF
        )

    @functools.partial(
        pl.kernel,
        out_type=tuple(outs) if with_deg else outs[0],
        mesh=plsc.VectorSubcoreMesh(core_axis_name="c", subcore_axis_name="s"),
        scratch_types=scratch,
    )
    def agg(*refs):
        _sc_agg_impl(with_deg, *refs)

    return agg


_sc_agg = _make_sc_agg(False)
_sc_agg_deg = _make_sc_agg(True)


ROWS_B = 2048
GRID = NP // ROWS_B
_BN_SCALE = 1.0 / (1.0 + 1e-5) ** 0.5


def _layer_math(x, m0, m1, d0, d1, w, b, gamma, beta):
    inv = 1.0 / jnp.maximum(d0 + d1, 1.0)
    cagg = (m0 + m1) * inv
    t = (jnp.dot(x, w[:HD, :], preferred_element_type=jnp.float32)
         + jnp.dot(cagg, w[HD:, :], preferred_element_type=jnp.float32) + b)
    nrm = jnp.sqrt(jnp.sum(t * t, axis=1, keepdims=True))
    t = t / jnp.maximum(nrm, 1e-12)
    t = jnp.maximum(t, 0.0)
    t = t * (gamma * _BN_SCALE) + beta
    return x + t


def _tc_embed_kernel(h_ref, emb_ref, o_ref):
    oh = (h_ref[...] == lax.broadcasted_iota(jnp.int32, (1, NA), 1))
    o_ref[...] = jnp.dot(oh.astype(jnp.float32), emb_ref[...],
                         preferred_element_type=jnp.float32)


def _tc_layer_kernel(x_ref, m0_ref, m1_ref, d0_ref, d1_ref,
                     w_ref, b_ref, g_ref, bt_ref, o_ref):
    o_ref[...] = _layer_math(x_ref[...], m0_ref[...], m1_ref[...],
                             d0_ref[...], d1_ref[...], w_ref[...],
                             b_ref[...], g_ref[...], bt_ref[...])


def _tc_final_kernel(x_ref, m0_ref, m1_ref, d0_ref, d1_ref,
                     w_ref, b_ref, g_ref, bt_ref,
                     mw1_ref, mb1_ref, mw2_ref, mb2_ref, mw3_ref, mb3_ref,
                     y_ref, acc_ref):
    i = pl.program_id(0)
    xn = _layer_math(x_ref[...], m0_ref[...], m1_ref[...],
                     d0_ref[...], d1_ref[...], w_ref[...],
                     b_ref[...], g_ref[...], bt_ref[...])
    rows = i * ROWS_B + lax.broadcasted_iota(jnp.int32, (ROWS_B, 1), 0)
    xn = jnp.where(rows < N_NODES, xn, 0.0)
    part = jnp.sum(xn, axis=0, keepdims=True)

    @pl.when(i == 0)
    def _():
        acc_ref[...] = jnp.zeros_like(acc_ref)

    acc_ref[...] += part

    @pl.when(i == GRID - 1)
    def _():
        hg = acc_ref[...] * (1.0 / N_NODES)
        y = jnp.maximum(jnp.dot(hg, mw1_ref[...],
                                preferred_element_type=jnp.float32)
                        + mb1_ref[...], 0.0)
        y = jnp.maximum(jnp.dot(y, mw2_ref[...],
                                preferred_element_type=jnp.float32)
                        + mb2_ref[...], 0.0)
        y_ref[...] = (jnp.dot(y, mw3_ref[...],
                              preferred_element_type=jnp.float32)
                      + mb3_ref[...])


_row_spec = pl.BlockSpec((ROWS_B, HD), lambda i: (i, 0))
_deg_spec = pl.BlockSpec((ROWS_B, 1), lambda i: (i, 0))


def _whole(shape):
    return pl.BlockSpec(shape, lambda i, _s=shape: tuple(0 for _ in _s))


def _tc_embed(h2, emb):
    return pl.pallas_call(
        _tc_embed_kernel,
        grid=(GRID,),
        in_specs=[_deg_spec, _whole((NA, HD))],
        out_specs=_row_spec,
        out_shape=jax.ShapeDtypeStruct((NP, HD), jnp.float32),
    )(h2, emb)


def _tc_layer(x, m0, m1, d0, d1, w, b, g, bt):
    return pl.pallas_call(
        _tc_layer_kernel,
        grid=(GRID,),
        in_specs=[_row_spec, _row_spec, _row_spec, _deg_spec, _deg_spec,
                  _whole((2 * HD, HD)), _whole((1, HD)), _whole((1, HD)),
                  _whole((1, HD))],
        out_specs=_row_spec,
        out_shape=jax.ShapeDtypeStruct((NP, HD), jnp.float32),
    )(x, m0, m1, d0, d1, w, b, g, bt)


def _tc_final(x, m0, m1, d0, d1, w, b, g, bt, mw1, mb1, mw2, mb2, mw3, mb3):
    return pl.pallas_call(
        _tc_final_kernel,
        grid=(GRID,),
        in_specs=[_row_spec, _row_spec, _row_spec, _deg_spec, _deg_spec,
                  _whole((2 * HD, HD)), _whole((1, HD)), _whole((1, HD)),
                  _whole((1, HD)),
                  _whole((HD, HD // 2)), _whole((1, HD // 2)),
                  _whole((HD // 2, HD // 4)), _whole((1, HD // 4)),
                  _whole((HD // 4, 1)), _whole((1, 1))],
        out_specs=_whole((1, 1)),
        out_shape=jax.ShapeDtypeStruct((1, 1), jnp.float32),
        scratch_shapes=[pltpu.VMEM((1, HD), jnp.float32)],
    )(x, m0, m1, d0, d1, w, b, g, bt, mw1, mb1, mw2, mb2, mw3, mb3)


def kernel(edge_index, h, e, emb, W, b, gamma, beta,
           mW1, mb1, mW2, mb2, mW3, mb3):
    del e
    src = edge_index[0].astype(jnp.int32)
    dst = edge_index[1].astype(jnp.int32)
    pad = N_NODES + (jnp.arange(EP - N_EDGES, dtype=jnp.int32)
                     % (NP - N_NODES))
    src_pad = jnp.concatenate([src, pad]).reshape(EROWS, ECHUNK)
    dst_pad = jnp.concatenate([dst, pad]).reshape(EROWS, ECHUNK)
    h2 = jnp.concatenate(
        [h.astype(jnp.int32),
         jnp.zeros((NP - N_NODES,), jnp.int32)]).reshape(NP, 1)

    x = _tc_embed(h2, emb)

    b2 = b.reshape(NL, 1, HD)
    g2 = gamma.reshape(NL, 1, HD)
    bt2 = beta.reshape(NL, 1, HD)

    m, degp = _sc_agg_deg(x, src_pad, dst_pad)
    d0 = degp[0].reshape(NP, 1)
    d1 = degp[1].reshape(NP, 1)
    x = _tc_layer(x, m[0], m[1], d0, d1, W[0], b2[0], g2[0], bt2[0])
    for l in range(1, NL - 1):
        m = _sc_agg(x, src_pad, dst_pad)
        x = _tc_layer(x, m[0], m[1], d0, d1, W[l], b2[l], g2[l], bt2[l])
    m = _sc_agg(x, src_pad, dst_pad)
    y = _tc_final(x, m[0], m[1], d0, d1, W[NL - 1], b2[NL - 1],
                  g2[NL - 1], bt2[NL - 1],
                  mW1, mb1.reshape(1, HD // 2), mW2, mb2.reshape(1, HD // 4),
                  mW3, mb3.reshape(1, 1))
    return y
